# baseline (speedup 1.0000x reference)
import jax
import jax.numpy as jnp
from jax import lax
from jax.experimental import pallas as pl
from jax.experimental.pallas import tpu as pltpu
from jax.experimental.pallas import tpu_sc as plsc

MAX_LENGTH = 200
DIM = 64
BATCH = 4096
SEQ = 200

_NC, _NS = 2, 16
_NW = _NC * _NS
_RPW = BATCH // _NW
_BUF_B = 4
_NCOPY = _RPW // _BUF_B


def _sc_body(tab_hbm, out_hbm, row_v, buf, sem):
    wid = lax.axis_index("s") * _NC + lax.axis_index("c")
    base = wid * _RPW

    pltpu.sync_copy(tab_hbm.at[pl.ds(MAX_LENGTH, 1)], row_v)
    chunks = [row_v[0, pl.ds(16 * j, 16)] for j in range(DIM // 16)]

    def fill(s, carry):
        for b in range(_BUF_B):
            for j in range(DIM // 16):
                buf[b, s, pl.ds(16 * j, 16)] = chunks[j]
        return carry

    lax.fori_loop(0, SEQ, fill, 0)

    copies = [
        pltpu.async_copy(buf, out_hbm.at[pl.ds(base + i * _BUF_B, _BUF_B)], sem)
        for i in range(_NCOPY)
    ]
    for c in copies:
        c.wait()


def kernel(inputs, kernel):
    del inputs
    k = pl.kernel(
        _sc_body,
        mesh=plsc.VectorSubcoreMesh(core_axis_name="c", subcore_axis_name="s"),
        out_type=jax.ShapeDtypeStruct((BATCH, SEQ, DIM), jnp.float32),
        scratch_types=[
            pltpu.VMEM((1, DIM), jnp.float32),
            pltpu.VMEM((_BUF_B, SEQ, DIM), jnp.float32),
            pltpu.SemaphoreType.DMA,
        ],
        compiler_params=pltpu.CompilerParams(use_tc_tiling_on_sc=True),
    )
    return k(kernel)


# speedup vs baseline: 1.7347x; 1.7347x over previous
import jax
import jax.numpy as jnp
from jax.experimental import pallas as pl
from jax.experimental.pallas import tpu as pltpu

MAX_LENGTH = 200
DIM = 64
BATCH = 4096
SEQ = 200

_BB = 256
_NCOPY = BATCH // _BB


def _fanout_kernel(tab_ref, out_ref, rowbuf, scratch, sems):
    row = tab_ref[MAX_LENGTH, :]
    for s in range(SEQ):
        rowbuf[:, pl.ds(s * DIM, DIM)] = row[None, :]
    scratch[...] = jnp.broadcast_to(rowbuf[...], scratch.shape)
    for i in range(_NCOPY):
        pltpu.make_async_copy(
            scratch, out_ref.at[pl.ds(i * _BB, _BB)], sems.at[i]).start()
    for i in range(_NCOPY):
        pltpu.make_async_copy(
            scratch, out_ref.at[pl.ds(i * _BB, _BB)], sems.at[i]).wait()


def kernel(inputs, kernel):
    del inputs
    flat = pl.pallas_call(
        _fanout_kernel,
        in_specs=[pl.BlockSpec(memory_space=pltpu.MemorySpace.VMEM)],
        out_specs=pl.BlockSpec(memory_space=pltpu.MemorySpace.HBM),
        out_shape=jax.ShapeDtypeStruct((BATCH, SEQ * DIM), jnp.float32),
        scratch_shapes=[
            pltpu.VMEM((1, SEQ * DIM), jnp.float32),
            pltpu.VMEM((_BB, SEQ * DIM), jnp.float32),
            pltpu.SemaphoreType.DMA((_NCOPY,)),
        ],
    )(kernel)
    return jnp.reshape(flat, (BATCH, SEQ, DIM))
